# baseline (speedup 1.0000x reference)
import jax
import jax.numpy as jnp
from jax import lax
from jax.experimental import pallas as pl
from jax.experimental.pallas import tpu as pltpu


def _dropblock_body(mask_ref, x_ref, o_ref, m_ref, *, bs, H, W, S):
    HW = H * W

    @pl.when(pl.program_id(0) == 0)
    def _():
        m = mask_ref[:]
        wcol = lax.broadcasted_iota(jnp.int32, (HW, 1), 0) & (W - 1)
        r = m
        for j in range(1, bs):
            sh = jnp.pad(m, ((j, 0), (0, 0)))[:HW, :]
            r = jnp.maximum(r, jnp.where(wcol >= j, sh, 0.0))
        b = r
        for i in range(1, bs):
            sh = jnp.pad(r, ((W * i, 0), (0, 0)))[:HW, :]
            b = jnp.maximum(b, sh)
        bm = 1.0 - b
        scale = jnp.float32(HW) / jnp.sum(bm)
        m_ref[:] = bm * scale

    off = (pl.program_id(0) % (HW // S)) * S
    o_ref[:] = x_ref[:] * m_ref[pl.ds(off, S), :]


def kernel(x, block_size, feat_size, drop_rate):
    B, C, H, W = x.shape
    bs = 7
    gamma = drop_rate / (block_size ** 2) * (
        (feat_size ** 2) / ((feat_size - block_size + 1) ** 2)
    )
    mkey = jax.random.fold_in(jax.random.key(0), 1)
    mask = jax.random.bernoulli(mkey, gamma, (H, W)).astype(jnp.float32)

    HW = H * W
    xt = x.transpose(0, 2, 3, 1).reshape(B * HW, C)

    S = HW // 2
    out = pl.pallas_call(
        lambda mask_ref, x_ref, o_ref, m_ref: _dropblock_body(
            mask_ref, x_ref, o_ref, m_ref, bs=bs, H=H, W=W, S=S
        ),
        grid=(B * HW // S,),
        in_specs=[
            pl.BlockSpec((HW, 1), lambda i: (0, 0)),
            pl.BlockSpec((S, C), lambda i: (i, 0)),
        ],
        out_specs=pl.BlockSpec((S, C), lambda i: (i, 0)),
        out_shape=jax.ShapeDtypeStruct((B * HW, C), x.dtype),
        scratch_shapes=[pltpu.VMEM((HW, 1), jnp.float32)],
        compiler_params=pltpu.CompilerParams(
            dimension_semantics=("arbitrary",),
        ),
    )(mask.reshape(HW, 1), xt)
    return out.reshape(B, H, W, C).transpose(0, 3, 1, 2)


# speedup vs baseline: 1.0962x; 1.0962x over previous
import jax
import jax.numpy as jnp
from jax import lax
from jax.experimental import pallas as pl
from jax.experimental.pallas import tpu as pltpu


def _dropblock_body(mask_ref, x_ref, o_ref, m_ref, *, bs, H, W, S):
    HW = H * W

    @pl.when(pl.program_id(0) == 0)
    def _():
        m = mask_ref[:]
        wcol = lax.broadcasted_iota(jnp.int32, (HW, 1), 0) & (W - 1)
        r = m
        for j in range(1, bs):
            sh = jnp.pad(m, ((j, 0), (0, 0)))[:HW, :]
            r = jnp.maximum(r, jnp.where(wcol >= j, sh, 0.0))
        b = r
        for i in range(1, bs):
            sh = jnp.pad(r, ((W * i, 0), (0, 0)))[:HW, :]
            b = jnp.maximum(b, sh)
        bm = 1.0 - b
        scale = jnp.float32(HW) / jnp.sum(bm)
        for k in range(max(1, S // HW)):
            m_ref[pl.ds(k * HW, HW), :] = bm * scale

    if S >= HW:
        o_ref[:] = x_ref[:] * m_ref[:]
    else:
        off = (pl.program_id(0) % (HW // S)) * S
        o_ref[:] = x_ref[:] * m_ref[pl.ds(off, S), :]


def kernel(x, block_size, feat_size, drop_rate):
    B, C, H, W = x.shape
    bs = 7
    gamma = drop_rate / (block_size ** 2) * (
        (feat_size ** 2) / ((feat_size - block_size + 1) ** 2)
    )
    mkey = jax.random.fold_in(jax.random.key(0), 1)
    mask = jax.random.bernoulli(mkey, gamma, (H, W)).astype(jnp.float32)

    HW = H * W
    xt = x.transpose(0, 2, 3, 1).reshape(B * HW, C)

    S = HW * 2
    out = pl.pallas_call(
        lambda mask_ref, x_ref, o_ref, m_ref: _dropblock_body(
            mask_ref, x_ref, o_ref, m_ref, bs=bs, H=H, W=W, S=S
        ),
        grid=(B * HW // S,),
        in_specs=[
            pl.BlockSpec((HW, 1), lambda i: (0, 0)),
            pl.BlockSpec((S, C), lambda i: (i, 0)),
        ],
        out_specs=pl.BlockSpec((S, C), lambda i: (i, 0)),
        out_shape=jax.ShapeDtypeStruct((B * HW, C), x.dtype),
        scratch_shapes=[pltpu.VMEM((max(S, HW), 1), jnp.float32)],
        compiler_params=pltpu.CompilerParams(
            dimension_semantics=("arbitrary",),
        ),
    )(mask.reshape(HW, 1), xt)
    return out.reshape(B, H, W, C).transpose(0, 3, 1, 2)
